# Initial kernel scaffold; baseline (speedup 1.0000x reference)
import functools

import jax
import jax.numpy as jnp
from jax import lax
from jax.experimental import pallas as pl
from jax.experimental.pallas import tpu as pltpu
from jax.experimental.pallas import tpu_sc as plsc

V = 1000000
H = 64
B = 4096
L = 200
P = 512
M = 32
EPS = 1e-12

ROWS = B * L


_NC, _NS = 2, 16
_NW = _NC * _NS
_ROWS_PER_W = ROWS // _NW
_CHUNK = 1024
_NCHUNKS = _ROWS_PER_W // _CHUNK


def _sc_gather(word_table, idx):
    mesh = plsc.VectorSubcoreMesh(core_axis_name="c", subcore_axis_name="s")

    @functools.partial(
        pl.kernel,
        mesh=mesh,
        out_type=jax.ShapeDtypeStruct((ROWS, H), jnp.float32),
        scratch_types=[
            pltpu.VMEM((_CHUNK,), jnp.int32),
            pltpu.VMEM((_CHUNK, H), jnp.float32),
            pltpu.SemaphoreType.DMA,
        ],
    )
    def k(table_hbm, idx_hbm, out_hbm, idx_v, rows_v, sem):
        wid = lax.axis_index("s") * _NC + lax.axis_index("c")
        wbase = wid * _ROWS_PER_W

        def body(c, carry):
            base = wbase + c * _CHUNK
            pltpu.sync_copy(idx_hbm.at[pl.ds(base, _CHUNK)], idx_v)
            pltpu.async_copy(table_hbm.at[idx_v], rows_v, sem).wait()
            pltpu.sync_copy(rows_v, out_hbm.at[pl.ds(base, _CHUNK)])
            return carry

        lax.fori_loop(0, _NCHUNKS, body, 0)

    return k(word_table, idx)


_SEQ_PER_BLK = 8
_RBLK = _SEQ_PER_BLK * L
_GRID = ROWS // _RBLK


def _tc_body(raw_ref, ipid_ref, pe_ref, item_ref, g_ref, b_ref, o_ref):
    x = raw_ref[...] + pe_ref[...]
    ids = ipid_ref[0]
    onehot = (ids == lax.broadcasted_iota(jnp.int32, (1, M), 1)).astype(jnp.float32)
    x = x + jnp.dot(onehot, item_ref[...], preferred_element_type=jnp.float32)
    mean = jnp.mean(x, axis=-1, keepdims=True)
    c = x - mean
    var = jnp.mean(c * c, axis=-1, keepdims=True)
    o_ref[...] = (c * lax.rsqrt(var + EPS)) * g_ref[...] + b_ref[...]


def _tc_ln(raw, ipid3, pe_tiled, item_table, gamma2, beta2):
    return pl.pallas_call(
        _tc_body,
        grid=(_GRID,),
        in_specs=[
            pl.BlockSpec((_RBLK, H), lambda i: (i, 0)),
            pl.BlockSpec((1, _RBLK, 1), lambda i: (i, 0, 0)),
            pl.BlockSpec((_RBLK, H), lambda i: (0, 0)),
            pl.BlockSpec((M, H), lambda i: (0, 0)),
            pl.BlockSpec((1, H), lambda i: (0, 0)),
            pl.BlockSpec((1, H), lambda i: (0, 0)),
        ],
        out_specs=pl.BlockSpec((_RBLK, H), lambda i: (i, 0)),
        out_shape=jax.ShapeDtypeStruct((ROWS, H), jnp.float32),
    )(raw, ipid3, pe_tiled, item_table, gamma2, beta2)


def kernel(input_ids, item_position_ids, word_table, pos_table, item_table, gamma, beta):
    idx = input_ids.reshape(ROWS).astype(jnp.int32)
    raw = _sc_gather(word_table, idx)
    ipid3 = item_position_ids.reshape(_GRID, _RBLK, 1).astype(jnp.int32)
    pe_tiled = jnp.tile(pos_table[:L], (_SEQ_PER_BLK, 1))
    out = _tc_ln(raw, ipid3, pe_tiled, item_table,
                 gamma.reshape(1, H), beta.reshape(1, H))
    return out.reshape(B, L, H)


# speedup vs baseline: 1.8172x; 1.8172x over previous
import functools

import jax
import jax.numpy as jnp
from jax import lax
from jax.experimental import pallas as pl
from jax.experimental.pallas import tpu as pltpu
from jax.experimental.pallas import tpu_sc as plsc

V = 1000000
H = 64
B = 4096
L = 200
P = 512
M = 32
EPS = 1e-12

ROWS = B * L


_NC, _NS = 2, 16
_NW = _NC * _NS
_ROWS_PER_W = ROWS // _NW
_CHUNK = 1024
_NCHUNKS = _ROWS_PER_W // _CHUNK


def _sc_gather(word_table, idx):
    mesh = plsc.VectorSubcoreMesh(core_axis_name="c", subcore_axis_name="s")

    @functools.partial(
        pl.kernel,
        mesh=mesh,
        out_type=jax.ShapeDtypeStruct((ROWS, H), jnp.float32),
        scratch_types=[
            pltpu.VMEM((_CHUNK,), jnp.int32),
            pltpu.VMEM((_CHUNK, H), jnp.float32),
            pltpu.SemaphoreType.DMA,
        ],
        compiler_params=pltpu.CompilerParams(use_tc_tiling_on_sc=False),
    )
    def k(table_hbm, idx_hbm, out_hbm, idx_v, rows_v, sem):
        wid = lax.axis_index("s") * _NC + lax.axis_index("c")
        wbase = wid * _ROWS_PER_W

        def body(c, carry):
            base = wbase + c * _CHUNK
            pltpu.sync_copy(idx_hbm.at[pl.ds(base, _CHUNK)], idx_v)
            pltpu.async_copy(table_hbm.at[idx_v], rows_v, sem).wait()
            pltpu.sync_copy(rows_v, out_hbm.at[pl.ds(base, _CHUNK)])
            return carry

        lax.fori_loop(0, _NCHUNKS, body, 0)

    return k(word_table, idx)


_SEQ_PER_BLK = 8
_RBLK = _SEQ_PER_BLK * L
_GRID = ROWS // _RBLK


def _tc_body(raw_ref, ipid_ref, pe_ref, item_ref, g_ref, b_ref, o_ref):
    x = raw_ref[...] + pe_ref[...]
    ids = ipid_ref[0]
    onehot = (ids == lax.broadcasted_iota(jnp.int32, (1, M), 1)).astype(jnp.float32)
    x = x + jnp.dot(onehot, item_ref[...], preferred_element_type=jnp.float32)
    mean = jnp.mean(x, axis=-1, keepdims=True)
    c = x - mean
    var = jnp.mean(c * c, axis=-1, keepdims=True)
    o_ref[...] = (c * lax.rsqrt(var + EPS)) * g_ref[...] + b_ref[...]


def _tc_ln(raw, ipid3, pe_tiled, item_table, gamma2, beta2):
    return pl.pallas_call(
        _tc_body,
        grid=(_GRID,),
        in_specs=[
            pl.BlockSpec((_RBLK, H), lambda i: (i, 0)),
            pl.BlockSpec((1, _RBLK, 1), lambda i: (i, 0, 0)),
            pl.BlockSpec((_RBLK, H), lambda i: (0, 0)),
            pl.BlockSpec((M, H), lambda i: (0, 0)),
            pl.BlockSpec((1, H), lambda i: (0, 0)),
            pl.BlockSpec((1, H), lambda i: (0, 0)),
        ],
        out_specs=pl.BlockSpec((_RBLK, H), lambda i: (i, 0)),
        out_shape=jax.ShapeDtypeStruct((ROWS, H), jnp.float32),
    )(raw, ipid3, pe_tiled, item_table, gamma2, beta2)


def kernel(input_ids, item_position_ids, word_table, pos_table, item_table, gamma, beta):
    idx = input_ids.reshape(ROWS).astype(jnp.int32)
    raw = _sc_gather(word_table, idx)
    ipid3 = item_position_ids.reshape(_GRID, _RBLK, 1).astype(jnp.int32)
    pe_tiled = jnp.tile(pos_table[:L], (_SEQ_PER_BLK, 1))
    out = _tc_ln(raw, ipid3, pe_tiled, item_table,
                 gamma.reshape(1, H), beta.reshape(1, H))
    return out.reshape(B, L, H)


# speedup vs baseline: 1.8596x; 1.0233x over previous
import functools

import jax
import jax.numpy as jnp
from jax import lax
from jax.experimental import pallas as pl
from jax.experimental.pallas import tpu as pltpu
from jax.experimental.pallas import tpu_sc as plsc

V = 1000000
H = 64
B = 4096
L = 200
P = 512
M = 32
EPS = 1e-12

ROWS = B * L
LPAD = 256


_NC, _NS = 2, 16
_NW = _NC * _NS
_SEQ_PER_W = B // _NW
_ROWS_PER_W = ROWS // _NW
_CHUNK = 1024
_NCHUNKS = _ROWS_PER_W // _CHUNK
_GRP = _CHUNK // 16


def _sc_gather(table128, ids256):
    mesh = plsc.VectorSubcoreMesh(core_axis_name="c", subcore_axis_name="s")

    @functools.partial(
        pl.kernel,
        mesh=mesh,
        out_type=jax.ShapeDtypeStruct((ROWS, H), jnp.float32),
        scratch_types=[
            pltpu.VMEM((_SEQ_PER_W, LPAD), jnp.int32),
            pltpu.VMEM((_CHUNK,), jnp.int32),
            pltpu.VMEM((_CHUNK, H), jnp.float32),
            pltpu.SemaphoreType.DMA,
        ],
        compiler_params=pltpu.CompilerParams(
            use_tc_tiling_on_sc=False, needs_layout_passes=False),
    )
    def k(table_hbm, ids_hbm, out_hbm, ids_v, widx_v, rows_v, sem):
        wid = lax.axis_index("s") * _NC + lax.axis_index("c")
        wbase = wid * _ROWS_PER_W
        tref = table_hbm

        pltpu.sync_copy(ids_hbm.at[pl.ds(wid * _SEQ_PER_W, _SEQ_PER_W)], ids_v)

        def chunk_body(c, carry):
            r0 = c * _CHUNK

            def grp_body(g, carry2):
                r = r0 + g * 16 + lax.iota(jnp.int32, 16)
                s = lax.div(r, jnp.int32(L))
                l = r - s * L
                vals = plsc.load_gather(ids_v, [s, l])
                widx_v[pl.ds(g * 16, 16)] = vals
                return carry2

            lax.fori_loop(0, _GRP, grp_body, 0)

            pltpu.async_copy(tref.at[widx_v], rows_v, sem).wait()
            pltpu.sync_copy(rows_v, out_hbm.at[pl.ds(wbase + r0, _CHUNK)])
            return carry

        lax.fori_loop(0, _NCHUNKS, chunk_body, 0)

    return k(table128, ids256)


_SEQ_PER_BLK = 16
_RBLK = _SEQ_PER_BLK * L
_GRID = B // _SEQ_PER_BLK


def _tc_body(raw_ref, ipid_ref, pe_ref, item_ref, g_ref, b_ref, o_ref):
    x = raw_ref[...] + pe_ref[...]
    ids = ipid_ref[0]
    onehot = (ids == lax.broadcasted_iota(jnp.int32, (1, M), 1)).astype(jnp.bfloat16)
    x = x + jnp.dot(onehot, item_ref[...], preferred_element_type=jnp.float32)
    mean = jnp.mean(x, axis=-1, keepdims=True)
    c = x - mean
    var = jnp.mean(c * c, axis=-1, keepdims=True)
    y = (c * lax.rsqrt(var + EPS)) * g_ref[...] + b_ref[...]
    o_ref[...] = y.reshape(_SEQ_PER_BLK, L, H)


def _tc_ln(raw, ipid3, pe_tiled, item_bf16, gamma2, beta2):
    return pl.pallas_call(
        _tc_body,
        grid=(_GRID,),
        in_specs=[
            pl.BlockSpec((_RBLK, H), lambda i: (i, 0)),
            pl.BlockSpec((1, _RBLK, 1), lambda i: (i, 0, 0)),
            pl.BlockSpec((_RBLK, H), lambda i: (0, 0)),
            pl.BlockSpec((M, H), lambda i: (0, 0)),
            pl.BlockSpec((1, H), lambda i: (0, 0)),
            pl.BlockSpec((1, H), lambda i: (0, 0)),
        ],
        out_specs=pl.BlockSpec((_SEQ_PER_BLK, L, H), lambda i: (i, 0, 0)),
        out_shape=jax.ShapeDtypeStruct((B, L, H), jnp.float32),
    )(raw, ipid3, pe_tiled, item_bf16, gamma2, beta2)


def kernel(input_ids, item_position_ids, word_table, pos_table, item_table, gamma, beta):
    ids256 = jnp.pad(input_ids.astype(jnp.int32), ((0, 0), (0, LPAD - L)))
    raw = _sc_gather(word_table, ids256)

    ipid3 = jnp.maximum(
        item_position_ids.reshape(_GRID, _RBLK, 1).astype(jnp.int32), 0)
    pe_tiled = jnp.tile(pos_table[:L], (_SEQ_PER_BLK, 1))
    out = _tc_ln(raw, ipid3, pe_tiled, item_table.astype(jnp.bfloat16),
                 gamma.reshape(1, H), beta.reshape(1, H))
    return out
